# Initial kernel scaffold; baseline (speedup 1.0000x reference)
import jax
import jax.numpy as jnp
from jax.experimental import pallas as pl


def kernel(x, edge_index, W1, b1, W2, b2):
    raise NotImplementedError("write your pallas kernel here")


# speedup vs baseline: 54.4554x; 54.4554x over previous
import functools

import jax
import jax.numpy as jnp
from jax import lax
from jax.experimental import pallas as pl
from jax.experimental.pallas import tpu as pltpu
from jax.experimental.pallas import tpu_sc as plsc

N = 10000
D_IN = 128
D_H = 16
E = 320000

NC, NS = 2, 16
NW = NC * NS
CHUNK = 128
NPAD = 10240
EPAD = 327680
CPT = EPAD // (NW * CHUNK)
ET = CPT * CHUNK
NPT = NPAD // NS

_mesh = plsc.VectorSubcoreMesh(
    core_axis_name="c", subcore_axis_name="s", num_cores=NC, num_subcores=NS)


@functools.partial(
    pl.kernel, mesh=_mesh,
    out_type=jax.ShapeDtypeStruct((NC * NPAD,), jnp.float32),
    scratch_types=[
        pltpu.VMEM((CPT, CHUNK), jnp.int32),
        pltpu.VMEM((CHUNK,), jnp.float32),
        pltpu.VMEM((NPT,), jnp.float32),
        pltpu.VMEM_SHARED((NPAD,), jnp.float32),
    ],
)
def _sc_deg(dst_hbm, ones_hbm, zeros_hbm, out_hbm, dstbuf, ones_v, stage, acc):
    c = lax.axis_index("c")
    s = lax.axis_index("s")
    wid = c * NS + s
    pltpu.sync_copy(ones_hbm, ones_v)
    pltpu.sync_copy(zeros_hbm.at[pl.ds(0, NPT)], stage)
    pltpu.sync_copy(stage, acc.at[pl.ds(s * NPT, NPT)])
    pltpu.sync_copy(dst_hbm.at[pl.ds(wid * CPT, CPT)], dstbuf)
    plsc.subcore_barrier()

    def chunk(j, carry):
        pltpu.sync_copy(ones_v, acc.at[dstbuf.at[j]], add=True)
        return carry

    lax.fori_loop(0, CPT, chunk, 0)
    plsc.subcore_barrier()
    pltpu.sync_copy(acc.at[pl.ds(s * NPT, NPT)], stage)
    pltpu.sync_copy(stage, out_hbm.at[pl.ds(c * NPAD + s * NPT, NPT)])


@functools.partial(
    pl.kernel, mesh=_mesh,
    out_type=jax.ShapeDtypeStruct((NC * NPAD, D_H), jnp.float32),
    scratch_types=[
        pltpu.VMEM((CPT, CHUNK), jnp.int32),
        pltpu.VMEM((CPT, CHUNK), jnp.int32),
        pltpu.VMEM((CHUNK, D_H), jnp.float32),
        pltpu.VMEM((CHUNK, D_H), jnp.float32),
        pltpu.VMEM_SHARED((NPAD, D_H), jnp.float32),
        pltpu.VMEM_SHARED((NPAD, D_H), jnp.float32),
        pltpu.SemaphoreType.DMA,
    ],
    compiler_params=pltpu.CompilerParams(use_tc_tiling_on_sc=False),
)
def _sc_agg1(q1_hbm, src_hbm, dst_hbm, zeros_hbm, out_hbm,
             srcbuf, dstbuf, rows, stage, qtab, acc, sem):
    c = lax.axis_index("c")
    s = lax.axis_index("s")
    wid = c * NS + s
    pltpu.sync_copy(zeros_hbm, stage)

    def zero(k, carry):
        pltpu.sync_copy(stage, acc.at[pl.ds(s * NPT + k * CHUNK, CHUNK)])
        return carry

    lax.fori_loop(0, NPT // CHUNK, zero, 0)

    def load_q(k, carry):
        off = s * NPT + k * CHUNK
        pltpu.sync_copy(q1_hbm.at[pl.ds(off, CHUNK)], rows)
        pltpu.sync_copy(rows, qtab.at[pl.ds(off, CHUNK)])
        return carry

    lax.fori_loop(0, NPT // CHUNK, load_q, 0)
    pltpu.sync_copy(src_hbm.at[pl.ds(wid * CPT, CPT)], srcbuf)
    pltpu.sync_copy(dst_hbm.at[pl.ds(wid * CPT, CPT)], dstbuf)
    plsc.subcore_barrier()

    def chunk(j, carry):
        pltpu.async_copy(qtab.at[srcbuf.at[j]], rows, sem).wait()
        pltpu.sync_copy(rows, acc.at[dstbuf.at[j]], add=True)
        return carry

    lax.fori_loop(0, CPT, chunk, 0)
    plsc.subcore_barrier()

    def wb(k, carry):
        off = s * NPT + k * CHUNK
        pltpu.sync_copy(acc.at[pl.ds(off, CHUNK)], rows)
        pltpu.sync_copy(rows, out_hbm.at[pl.ds(c * NPAD + off, CHUNK)])
        return carry

    lax.fori_loop(0, NPT // CHUNK, wb, 0)


@functools.partial(
    pl.kernel, mesh=_mesh,
    out_type=jax.ShapeDtypeStruct((NC * NPAD,), jnp.float32),
    scratch_types=[
        pltpu.VMEM((CPT, CHUNK), jnp.int32),
        pltpu.VMEM((CPT, CHUNK), jnp.int32),
        pltpu.VMEM((CHUNK,), jnp.float32),
        pltpu.VMEM((NPT,), jnp.float32),
        pltpu.VMEM_SHARED((NPAD,), jnp.float32),
        pltpu.VMEM_SHARED((NPAD,), jnp.float32),
        pltpu.SemaphoreType.DMA,
    ],
)
def _sc_agg2(q2_hbm, src_hbm, dst_hbm, zeros_hbm, out_hbm,
             srcbuf, dstbuf, vals, stage, qtab, acc, sem):
    c = lax.axis_index("c")
    s = lax.axis_index("s")
    wid = c * NS + s
    pltpu.sync_copy(zeros_hbm.at[pl.ds(0, NPT)], stage)
    pltpu.sync_copy(stage, acc.at[pl.ds(s * NPT, NPT)])
    pltpu.sync_copy(q2_hbm.at[pl.ds(s * NPT, NPT)], stage)
    pltpu.sync_copy(stage, qtab.at[pl.ds(s * NPT, NPT)])
    pltpu.sync_copy(src_hbm.at[pl.ds(wid * CPT, CPT)], srcbuf)
    pltpu.sync_copy(dst_hbm.at[pl.ds(wid * CPT, CPT)], dstbuf)
    plsc.subcore_barrier()

    def chunk(j, carry):
        pltpu.async_copy(qtab.at[srcbuf.at[j]], vals, sem).wait()
        pltpu.sync_copy(vals, acc.at[dstbuf.at[j]], add=True)
        return carry

    lax.fori_loop(0, CPT, chunk, 0)
    plsc.subcore_barrier()
    pltpu.sync_copy(acc.at[pl.ds(s * NPT, NPT)], stage)
    pltpu.sync_copy(stage, out_hbm.at[pl.ds(c * NPAD + s * NPT, NPT)])


def _tca_body(x_ref, w1_ref, degp_ref, q1_ref, dinv_ref):
    deg = degp_ref[0] + degp_ref[1] + 1.0
    dinv = lax.rsqrt(deg)
    xw = jnp.dot(x_ref[...], w1_ref[...], preferred_element_type=jnp.float32)
    q1_ref[...] = xw * dinv
    dinv_ref[...] = dinv


_tca = pl.pallas_call(
    _tca_body,
    out_shape=[jax.ShapeDtypeStruct((NPAD, D_H), jnp.float32),
               jax.ShapeDtypeStruct((NPAD, 1), jnp.float32)],
)


def _tcb_body(aggp_ref, q1_ref, dinv_ref, b1_ref, w2_ref, q2_ref):
    agg = aggp_ref[0] + aggp_ref[1] + q1_ref[...]
    h = jnp.maximum(dinv_ref[...] * agg + b1_ref[...], 0.0)
    q2 = jnp.dot(h, w2_ref[...], preferred_element_type=jnp.float32)
    q2_ref[...] = q2 * dinv_ref[...]


_tcb = pl.pallas_call(
    _tcb_body,
    out_shape=jax.ShapeDtypeStruct((NPAD, 1), jnp.float32),
)


def _tcc_body(a2p_ref, q2_ref, dinv_ref, b2_ref, out_ref):
    out_ref[...] = dinv_ref[...] * (a2p_ref[0] + a2p_ref[1] + q2_ref[...]) \
        + b2_ref[...]


_tcc = pl.pallas_call(
    _tcc_body,
    out_shape=jax.ShapeDtypeStruct((NPAD, 1), jnp.float32),
)


def kernel(x, edge_index, W1, b1, W2, b2):
    pad = (jnp.arange(EPAD - E, dtype=jnp.int32) % (NPAD - N)) + N
    srcp = jnp.concatenate([edge_index[0], pad]).reshape(EPAD // CHUNK, CHUNK)
    dstp = jnp.concatenate([edge_index[1], pad]).reshape(EPAD // CHUNK, CHUNK)
    xpad = jnp.pad(x, ((0, NPAD - N), (0, 0)))

    ones_c = jnp.ones((CHUNK,), jnp.float32)
    zeros_r = jnp.zeros((CHUNK, D_H), jnp.float32)
    zeros_n = jnp.zeros((NPT,), jnp.float32)

    degp = _sc_deg(dstp, ones_c, zeros_n)
    q1, dinv = _tca(xpad, W1, degp.reshape(NC, NPAD, 1))
    aggp = _sc_agg1(q1, srcp, dstp, zeros_r).reshape(NC, NPAD, D_H)
    q2 = _tcb(aggp, q1, dinv, b1.reshape(1, D_H), W2)
    a2p = _sc_agg2(q2.reshape(NPAD), srcp, dstp, zeros_n)
    out = _tcc(a2p.reshape(NC, NPAD, 1), q2, dinv, b2.reshape(1, 1))
    return out[:N]


# speedup vs baseline: 62.3185x; 1.1444x over previous
import functools

import jax
import jax.numpy as jnp
from jax import lax
from jax.experimental import pallas as pl
from jax.experimental.pallas import tpu as pltpu
from jax.experimental.pallas import tpu_sc as plsc

N = 10000
D_IN = 128
D_H = 16
E = 320000

NC, NS = 2, 16
NW = NC * NS
CHUNK = 128
NPAD = 10240
EPAD = 327680
CPT = EPAD // (NW * CHUNK)
ET = CPT * CHUNK
NPT = NPAD // NS
GR = 8
NGR = CPT // GR

_mesh = plsc.VectorSubcoreMesh(
    core_axis_name="c", subcore_axis_name="s", num_cores=NC, num_subcores=NS)


@functools.partial(
    pl.kernel, mesh=_mesh,
    out_type=jax.ShapeDtypeStruct((NC * NPAD,), jnp.float32),
    scratch_types=[
        pltpu.VMEM((CPT, CHUNK), jnp.int32),
        pltpu.VMEM((CHUNK,), jnp.float32),
        pltpu.VMEM((NPT,), jnp.float32),
        pltpu.VMEM_SHARED((NPAD,), jnp.float32),
        pltpu.SemaphoreType.DMA,
    ],
)
def _sc_deg(dst_hbm, ones_hbm, zeros_hbm, out_hbm, dstbuf, ones_v, stage, acc,
            ssem):
    c = lax.axis_index("c")
    s = lax.axis_index("s")
    wid = c * NS + s
    pltpu.sync_copy(ones_hbm, ones_v)
    pltpu.sync_copy(zeros_hbm.at[pl.ds(0, NPT)], stage)
    pltpu.sync_copy(stage, acc.at[pl.ds(s * NPT, NPT)])
    pltpu.sync_copy(dst_hbm.at[pl.ds(wid * CPT, CPT)], dstbuf)
    plsc.subcore_barrier()

    def group(g, carry):
        base = g * GR
        scs = [pltpu.async_copy(ones_v, acc.at[dstbuf.at[base + k]], ssem,
                                add=True)
               for k in range(GR)]
        for d in scs:
            d.wait()
        return carry

    lax.fori_loop(0, NGR, group, 0)
    plsc.subcore_barrier()
    pltpu.sync_copy(acc.at[pl.ds(s * NPT, NPT)], stage)
    pltpu.sync_copy(stage, out_hbm.at[pl.ds(c * NPAD + s * NPT, NPT)])


@functools.partial(
    pl.kernel, mesh=_mesh,
    out_type=jax.ShapeDtypeStruct((NC * NPAD, D_H), jnp.float32),
    scratch_types=[
        pltpu.VMEM((CPT, CHUNK), jnp.int32),
        pltpu.VMEM((CPT, CHUNK), jnp.int32),
        pltpu.VMEM((GR, CHUNK, D_H), jnp.float32),
        pltpu.VMEM((CHUNK, D_H), jnp.float32),
        pltpu.VMEM_SHARED((NPAD, D_H), jnp.float32),
        pltpu.VMEM_SHARED((NPAD, D_H), jnp.float32),
        pltpu.SemaphoreType.DMA,
        pltpu.SemaphoreType.DMA,
    ],
    compiler_params=pltpu.CompilerParams(use_tc_tiling_on_sc=False),
)
def _sc_agg1(q1_hbm, src_hbm, dst_hbm, zeros_hbm, out_hbm,
             srcbuf, dstbuf, rows, stage, qtab, acc, gsem, ssem):
    c = lax.axis_index("c")
    s = lax.axis_index("s")
    wid = c * NS + s
    pltpu.sync_copy(zeros_hbm, stage)

    def zero(k, carry):
        pltpu.sync_copy(stage, acc.at[pl.ds(s * NPT + k * CHUNK, CHUNK)])
        return carry

    lax.fori_loop(0, NPT // CHUNK, zero, 0)

    def load_q(k, carry):
        off = s * NPT + k * CHUNK
        pltpu.sync_copy(q1_hbm.at[pl.ds(off, CHUNK)], stage)
        pltpu.sync_copy(stage, qtab.at[pl.ds(off, CHUNK)])
        return carry

    lax.fori_loop(0, NPT // CHUNK, load_q, 0)
    pltpu.sync_copy(src_hbm.at[pl.ds(wid * CPT, CPT)], srcbuf)
    pltpu.sync_copy(dst_hbm.at[pl.ds(wid * CPT, CPT)], dstbuf)
    plsc.subcore_barrier()

    def group(g, carry):
        base = g * GR
        gds = [pltpu.async_copy(qtab.at[srcbuf.at[base + k]], rows.at[k], gsem)
               for k in range(GR)]
        scs = []
        for k in range(GR):
            gds[k].wait()
            scs.append(pltpu.async_copy(rows.at[k], acc.at[dstbuf.at[base + k]],
                                        ssem, add=True))
        for d in scs:
            d.wait()
        return carry

    lax.fori_loop(0, NGR, group, 0)
    plsc.subcore_barrier()

    def wb(k, carry):
        off = s * NPT + k * CHUNK
        pltpu.sync_copy(acc.at[pl.ds(off, CHUNK)], stage)
        pltpu.sync_copy(stage, out_hbm.at[pl.ds(c * NPAD + off, CHUNK)])
        return carry

    lax.fori_loop(0, NPT // CHUNK, wb, 0)


@functools.partial(
    pl.kernel, mesh=_mesh,
    out_type=jax.ShapeDtypeStruct((NC * NPAD,), jnp.float32),
    scratch_types=[
        pltpu.VMEM((CPT, CHUNK), jnp.int32),
        pltpu.VMEM((CPT, CHUNK), jnp.int32),
        pltpu.VMEM((GR, CHUNK), jnp.float32),
        pltpu.VMEM((NPT,), jnp.float32),
        pltpu.VMEM_SHARED((NPAD,), jnp.float32),
        pltpu.VMEM_SHARED((NPAD,), jnp.float32),
        pltpu.SemaphoreType.DMA,
        pltpu.SemaphoreType.DMA,
    ],
)
def _sc_agg2(q2_hbm, src_hbm, dst_hbm, zeros_hbm, out_hbm,
             srcbuf, dstbuf, vals, stage, qtab, acc, gsem, ssem):
    c = lax.axis_index("c")
    s = lax.axis_index("s")
    wid = c * NS + s
    pltpu.sync_copy(zeros_hbm.at[pl.ds(0, NPT)], stage)
    pltpu.sync_copy(stage, acc.at[pl.ds(s * NPT, NPT)])
    pltpu.sync_copy(q2_hbm.at[pl.ds(s * NPT, NPT)], stage)
    pltpu.sync_copy(stage, qtab.at[pl.ds(s * NPT, NPT)])
    pltpu.sync_copy(src_hbm.at[pl.ds(wid * CPT, CPT)], srcbuf)
    pltpu.sync_copy(dst_hbm.at[pl.ds(wid * CPT, CPT)], dstbuf)
    plsc.subcore_barrier()

    def group(g, carry):
        base = g * GR
        gds = [pltpu.async_copy(qtab.at[srcbuf.at[base + k]], vals.at[k], gsem)
               for k in range(GR)]
        scs = []
        for k in range(GR):
            gds[k].wait()
            scs.append(pltpu.async_copy(vals.at[k], acc.at[dstbuf.at[base + k]],
                                        ssem, add=True))
        for d in scs:
            d.wait()
        return carry

    lax.fori_loop(0, NGR, group, 0)
    plsc.subcore_barrier()
    pltpu.sync_copy(acc.at[pl.ds(s * NPT, NPT)], stage)
    pltpu.sync_copy(stage, out_hbm.at[pl.ds(c * NPAD + s * NPT, NPT)])


def _tca_body(x_ref, w1_ref, degp_ref, q1_ref, dinv_ref):
    deg = degp_ref[0] + degp_ref[1] + 1.0
    dinv = lax.rsqrt(deg)
    xw = jnp.dot(x_ref[...], w1_ref[...], preferred_element_type=jnp.float32)
    q1_ref[...] = xw * dinv
    dinv_ref[...] = dinv


_tca = pl.pallas_call(
    _tca_body,
    out_shape=[jax.ShapeDtypeStruct((NPAD, D_H), jnp.float32),
               jax.ShapeDtypeStruct((NPAD, 1), jnp.float32)],
)


def _tcb_body(aggp_ref, q1_ref, dinv_ref, b1_ref, w2_ref, q2_ref):
    agg = aggp_ref[0] + aggp_ref[1] + q1_ref[...]
    h = jnp.maximum(dinv_ref[...] * agg + b1_ref[...], 0.0)
    q2 = jnp.dot(h, w2_ref[...], preferred_element_type=jnp.float32)
    q2_ref[...] = q2 * dinv_ref[...]


_tcb = pl.pallas_call(
    _tcb_body,
    out_shape=jax.ShapeDtypeStruct((NPAD, 1), jnp.float32),
)


def _tcc_body(a2p_ref, q2_ref, dinv_ref, b2_ref, out_ref):
    out_ref[...] = dinv_ref[...] * (a2p_ref[0] + a2p_ref[1] + q2_ref[...]) \
        + b2_ref[...]


_tcc = pl.pallas_call(
    _tcc_body,
    out_shape=jax.ShapeDtypeStruct((NPAD, 1), jnp.float32),
)


def kernel(x, edge_index, W1, b1, W2, b2):
    pad = (jnp.arange(EPAD - E, dtype=jnp.int32) % (NPAD - N)) + N
    srcp = jnp.concatenate([edge_index[0], pad]).reshape(EPAD // CHUNK, CHUNK)
    dstp = jnp.concatenate([edge_index[1], pad]).reshape(EPAD // CHUNK, CHUNK)
    xpad = jnp.pad(x, ((0, NPAD - N), (0, 0)))

    ones_c = jnp.ones((CHUNK,), jnp.float32)
    zeros_r = jnp.zeros((CHUNK, D_H), jnp.float32)
    zeros_n = jnp.zeros((NPT,), jnp.float32)

    degp = _sc_deg(dstp, ones_c, zeros_n)
    q1, dinv = _tca(xpad, W1, degp.reshape(NC, NPAD, 1))
    aggp = _sc_agg1(q1, srcp, dstp, zeros_r).reshape(NC, NPAD, D_H)
    q2 = _tcb(aggp, q1, dinv, b1.reshape(1, D_H), W2)
    a2p = _sc_agg2(q2.reshape(NPAD), srcp, dstp, zeros_n)
    out = _tcc(a2p.reshape(NC, NPAD, 1), q2, dinv, b2.reshape(1, 1))
    return out[:N]


# speedup vs baseline: 77.2128x; 1.2390x over previous
import functools

import jax
import jax.numpy as jnp
from jax import lax
from jax.experimental import pallas as pl
from jax.experimental.pallas import tpu as pltpu
from jax.experimental.pallas import tpu_sc as plsc

N = 10000
D_IN = 128
D_H = 16
E = 320000

NC, NS = 2, 16
NW = NC * NS
CHUNK = 128
NPAD = 10240
EPAD = 327680
CPT = EPAD // (NW * CHUNK)
NPT = NPAD // NS
GR = 8
NGR = CPT // GR
RC = 80

_mesh = plsc.VectorSubcoreMesh(
    core_axis_name="c", subcore_axis_name="s", num_cores=NC, num_subcores=NS)

_untiled = pltpu.CompilerParams(use_tc_tiling_on_sc=False,
                               needs_layout_passes=False)


@functools.partial(
    pl.kernel, mesh=_mesh,
    out_type=jax.ShapeDtypeStruct((NC * NPAD,), jnp.float32),
    scratch_types=[
        pltpu.VMEM((CPT, CHUNK), jnp.int32),
        pltpu.VMEM((CHUNK,), jnp.float32),
        pltpu.VMEM((NPT,), jnp.float32),
        pltpu.VMEM_SHARED((NPAD,), jnp.float32),
        pltpu.SemaphoreType.DMA,
    ],
)
def _sc_deg(dst_hbm, ones_hbm, zeros_hbm, out_hbm, dstbuf, ones_v, stage, acc,
            ssem):
    c = lax.axis_index("c")
    s = lax.axis_index("s")
    wid = c * NS + s
    pltpu.sync_copy(ones_hbm, ones_v)
    pltpu.sync_copy(zeros_hbm.at[pl.ds(0, NPT)], stage)
    pltpu.sync_copy(stage, acc.at[pl.ds(s * NPT, NPT)])
    pltpu.sync_copy(dst_hbm.at[pl.ds(wid * CPT, CPT)], dstbuf)
    plsc.subcore_barrier()

    def group(g, carry):
        base = g * GR
        scs = [pltpu.async_copy(ones_v, acc.at[dstbuf.at[base + k]], ssem,
                                add=True)
               for k in range(GR)]
        for d in scs:
            d.wait()
        return carry

    lax.fori_loop(0, NGR, group, 0)
    plsc.subcore_barrier()
    pltpu.sync_copy(acc.at[pl.ds(s * NPT, NPT)], stage)
    pltpu.sync_copy(stage, out_hbm.at[pl.ds(c * NPAD + s * NPT, NPT)])


@functools.partial(
    pl.kernel, mesh=_mesh,
    out_type=jax.ShapeDtypeStruct((NC * NPAD, D_H), jnp.float32),
    scratch_types=[
        pltpu.VMEM((CPT, CHUNK), jnp.int32),
        pltpu.VMEM((CPT, CHUNK), jnp.int32),
        pltpu.VMEM((GR, CHUNK, D_H), jnp.float32),
        pltpu.VMEM((CHUNK, D_H), jnp.float32),
        pltpu.VMEM((RC, D_H), jnp.float32),
        pltpu.VMEM((NPT,), jnp.float32),
        pltpu.VMEM((NPT // CHUNK, CHUNK), jnp.int32),
        pltpu.VMEM_SHARED((NPAD, D_H), jnp.float32),
        pltpu.VMEM_SHARED((NPAD, D_H), jnp.float32),
        pltpu.SemaphoreType.DMA,
        pltpu.SemaphoreType.DMA,
    ],
    compiler_params=_untiled,
)
def _sc_agg1(xw_hbm, dinv_hbm, src_hbm, dst_hbm, zeros_hbm, iota_hbm,
             out_hbm, srcbuf, dstbuf, rows, zstage, stage, dinvbuf, selfidx,
             qtab, acc, gsem, ssem):
    c = lax.axis_index("c")
    s = lax.axis_index("s")
    wid = c * NS + s
    pltpu.sync_copy(zeros_hbm, zstage)

    def zero(k, carry):
        pltpu.sync_copy(zstage, acc.at[pl.ds(s * NPT + k * CHUNK, CHUNK)])
        return carry

    lax.fori_loop(0, NPT // CHUNK, zero, 0)
    pltpu.sync_copy(dinv_hbm.at[pl.ds(s * NPT, NPT)], dinvbuf)
    pltpu.sync_copy(iota_hbm.at[pl.ds(s * (NPT // CHUNK), NPT // CHUNK)],
                    selfidx)

    def stage_rows(nch):
        def ch(k, carry):
            off = s * NPT + k * RC
            pltpu.sync_copy(xw_hbm.at[pl.ds(off, RC)], stage)

            def grp(g, carry2):
                dvec = dinvbuf[pl.ds(k * RC + g * 16, 16)]
                for i in range(16):
                    r = g * 16 + i
                    stage[r, :] = stage[r, :] * dvec[i]
                return carry2

            lax.fori_loop(0, RC // 16, grp, 0)
            pltpu.sync_copy(stage, qtab.at[pl.ds(off, RC)])
            return carry

        lax.fori_loop(0, nch, ch, 0)

    @pl.when(s < NS - 1)
    def _():
        stage_rows(NPT // RC)

    @pl.when(s == NS - 1)
    def _():
        stage_rows((N - (NS - 1) * NPT) // RC)

    pltpu.sync_copy(src_hbm.at[pl.ds(wid * CPT, CPT)], srcbuf)
    pltpu.sync_copy(dst_hbm.at[pl.ds(wid * CPT, CPT)], dstbuf)
    plsc.subcore_barrier()

    def group(g, carry):
        base = g * GR
        gds = [pltpu.async_copy(qtab.at[srcbuf.at[base + k]], rows.at[k], gsem)
               for k in range(GR)]
        scs = []
        for k in range(GR):
            gds[k].wait()
            scs.append(pltpu.async_copy(rows.at[k], acc.at[dstbuf.at[base + k]],
                                        ssem, add=True))
        for d in scs:
            d.wait()
        return carry

    lax.fori_loop(0, NGR, group, 0)
    plsc.subcore_barrier()

    @pl.when(c == 0)
    def _():
        def selfadd(k, carry):
            off = s * NPT + k * CHUNK
            pltpu.sync_copy(qtab.at[pl.ds(off, CHUNK)], zstage)
            pltpu.sync_copy(zstage, acc.at[selfidx.at[k]], add=True)
            return carry

        lax.fori_loop(0, NPT // CHUNK, selfadd, 0)

    plsc.subcore_barrier()

    def wb(k, carry):
        off = s * NPT + k * RC
        pltpu.sync_copy(acc.at[pl.ds(off, RC)], stage)

        def grp(g, carry2):
            dvec = dinvbuf[pl.ds(k * RC + g * 16, 16)]
            for i in range(16):
                r = g * 16 + i
                stage[r, :] = stage[r, :] * dvec[i]
            return carry2

        lax.fori_loop(0, RC // 16, grp, 0)
        pltpu.sync_copy(stage, out_hbm.at[pl.ds(c * NPAD + off, RC)])
        return carry

    lax.fori_loop(0, NPT // RC, wb, 0)


@functools.partial(
    pl.kernel, mesh=_mesh,
    out_type=jax.ShapeDtypeStruct((NC * NPAD,), jnp.float32),
    scratch_types=[
        pltpu.VMEM((CPT, CHUNK), jnp.int32),
        pltpu.VMEM((CPT, CHUNK), jnp.int32),
        pltpu.VMEM((GR, CHUNK), jnp.float32),
        pltpu.VMEM((RC, D_H), jnp.float32),
        pltpu.VMEM((16, 16), jnp.float32),
        pltpu.VMEM((NPT,), jnp.float32),
        pltpu.VMEM((NPT,), jnp.float32),
        pltpu.VMEM((NPT,), jnp.float32),
        pltpu.VMEM((NPT // CHUNK, CHUNK), jnp.int32),
        pltpu.VMEM_SHARED((NPAD,), jnp.float32),
        pltpu.VMEM_SHARED((NPAD,), jnp.float32),
        pltpu.SemaphoreType.DMA,
        pltpu.SemaphoreType.DMA,
    ],
    compiler_params=_untiled,
)
def _sc_agg2(hw_hbm, dinv_hbm, src_hbm, dst_hbm, zeros_hbm, eye_hbm,
             iota_hbm, out_hbm, srcbuf, dstbuf, vals, rstage, ohbuf, qbuf,
             dinvbuf, stage, selfidx, qtab, acc, gsem, ssem):
    c = lax.axis_index("c")
    s = lax.axis_index("s")
    wid = c * NS + s
    pltpu.sync_copy(zeros_hbm.at[pl.ds(0, NPT)], stage)
    pltpu.sync_copy(stage, acc.at[pl.ds(s * NPT, NPT)])
    pltpu.sync_copy(dinv_hbm.at[pl.ds(s * NPT, NPT)], dinvbuf)
    pltpu.sync_copy(iota_hbm.at[pl.ds(s * (NPT // CHUNK), NPT // CHUNK)],
                    selfidx)

    pltpu.sync_copy(eye_hbm, ohbuf)

    def qch(k, carry):
        off = s * NPT + k * RC
        pltpu.sync_copy(hw_hbm.at[pl.ds(off, RC)], rstage)

        def grp(g, carry2):
            dvec = dinvbuf[pl.ds(k * RC + g * 16, 16)]
            v = jnp.zeros((16,), jnp.float32)
            for i in range(16):
                v = v + jnp.sum(rstage[g * 16 + i, :]) * ohbuf[i, :]
            qbuf[pl.ds(k * RC + g * 16, 16)] = v * dvec
            return carry2

        lax.fori_loop(0, RC // 16, grp, 0)
        return carry

    lax.fori_loop(0, NPT // RC, qch, 0)
    pltpu.sync_copy(qbuf, qtab.at[pl.ds(s * NPT, NPT)])
    pltpu.sync_copy(src_hbm.at[pl.ds(wid * CPT, CPT)], srcbuf)
    pltpu.sync_copy(dst_hbm.at[pl.ds(wid * CPT, CPT)], dstbuf)
    plsc.subcore_barrier()

    def group(g, carry):
        base = g * GR
        gds = [pltpu.async_copy(qtab.at[srcbuf.at[base + k]], vals.at[k], gsem)
               for k in range(GR)]
        scs = []
        for k in range(GR):
            gds[k].wait()
            scs.append(pltpu.async_copy(vals.at[k], acc.at[dstbuf.at[base + k]],
                                        ssem, add=True))
        for d in scs:
            d.wait()
        return carry

    lax.fori_loop(0, NGR, group, 0)
    plsc.subcore_barrier()

    @pl.when(c == 0)
    def _():
        def selfadd(k, carry):
            off = s * NPT + k * CHUNK
            pltpu.sync_copy(qtab.at[pl.ds(off, CHUNK)], vals.at[0])
            pltpu.sync_copy(vals.at[0], acc.at[selfidx.at[k]], add=True)
            return carry

        lax.fori_loop(0, NPT // CHUNK, selfadd, 0)

    plsc.subcore_barrier()
    pltpu.sync_copy(acc.at[pl.ds(s * NPT, NPT)], stage)
    pltpu.sync_copy(stage, out_hbm.at[pl.ds(c * NPAD + s * NPT, NPT)])


def _tca_body(x_ref, w1_ref, degp_ref, xw_ref, dinvp_ref):
    xw_ref[...] = jnp.dot(x_ref[...], w1_ref[...],
                          preferred_element_type=jnp.float32)
    dinvp_ref[...] = lax.rsqrt(degp_ref[0] + degp_ref[1] + 1.0)


_tca = pl.pallas_call(
    _tca_body,
    out_shape=[jax.ShapeDtypeStruct((N, D_H), jnp.float32),
               jax.ShapeDtypeStruct((NPAD // 128, 128), jnp.float32)],
)


def _tcb_body(sp_ref, b1_ref, w2r_ref, hw_ref):
    p = sp_ref[pl.ds(0, NPAD), :] + sp_ref[pl.ds(NPAD, NPAD), :]
    h = jnp.maximum(p + b1_ref[...], 0.0)
    hw_ref[...] = h * w2r_ref[...]


_tcb = pl.pallas_call(
    _tcb_body,
    out_shape=jax.ShapeDtypeStruct((NPAD, D_H), jnp.float32),
)


def _tcc_body(a2p_ref, dinvp_ref, b2_ref, out_ref):
    out_ref[...] = dinvp_ref[...] * (a2p_ref[0] + a2p_ref[1]) + b2_ref[...]


_tcc = pl.pallas_call(
    _tcc_body,
    out_shape=jax.ShapeDtypeStruct((NPAD // 128, 128), jnp.float32),
)


def kernel(x, edge_index, W1, b1, W2, b2):
    pad = (jnp.arange(EPAD - E, dtype=jnp.int32) % (NPAD - N)) + N
    srcp = jnp.concatenate([edge_index[0], pad]).reshape(EPAD // CHUNK, CHUNK)
    dstp = jnp.concatenate([edge_index[1], pad]).reshape(EPAD // CHUNK, CHUNK)

    ones_c = jnp.ones((CHUNK,), jnp.float32)
    zeros_r = jnp.zeros((CHUNK, D_H), jnp.float32)
    zeros_n = jnp.zeros((NPT,), jnp.float32)

    degp = _sc_deg(dstp, ones_c, zeros_n)
    xw, dinvp = _tca(x, W1, degp.reshape(NC, NPAD // 128, 128))
    dinv = dinvp.reshape(NPAD)
    iota2d = jnp.arange(NPAD, dtype=jnp.int32).reshape(NPAD // CHUNK, CHUNK)
    sp = _sc_agg1(xw, dinv, srcp, dstp, zeros_r, iota2d)
    hw = _tcb(sp, b1.reshape(1, D_H), W2.reshape(1, D_H))
    eye16 = jnp.eye(D_H, dtype=jnp.float32)
    a2p = _sc_agg2(hw, dinv, srcp, dstp, zeros_n, eye16, iota2d)
    outp = _tcc(a2p.reshape(NC, NPAD // 128, 128), dinvp, b2.reshape(1, 1))
    return outp.reshape(NPAD, 1)[:N]
